# Initial kernel scaffold; baseline (speedup 1.0000x reference)
import jax
import jax.numpy as jnp
from jax.experimental import pallas as pl


def kernel(X_fixed, W_ih, W_hh, b_ih, b_hh, W_fc, b_fc):
    raise NotImplementedError("write your pallas kernel here")


# speedup vs baseline: 652.2634x; 652.2634x over previous
import jax
import jax.numpy as jnp
from jax.experimental import pallas as pl

S, B, D, H, T = 2048, 16, 256, 128, 10


def _fwd(x0_ref, wih_t_ref, bih_ref, bhh_ref, wfc_t_ref, bfc_ref, o_ref):
    x0 = x0_ref[0]
    gi = jnp.dot(x0, wih_t_ref[...], preferred_element_type=jnp.float32)
    gi = gi + bih_ref[...]
    bhh = bhh_ref[...]
    r = jax.nn.sigmoid(gi[:, :H] + bhh[:, :H])
    z = jax.nn.sigmoid(gi[:, H:2 * H] + bhh[:, H:2 * H])
    n = jnp.tanh(gi[:, 2 * H:] + r * bhh[:, 2 * H:])
    h1 = (1.0 - z) * n
    logits = jnp.dot(h1, wfc_t_ref[...], preferred_element_type=jnp.float32)
    logits = logits + bfc_ref[...]
    m = jnp.max(logits, axis=1, keepdims=True)
    s = logits - m
    lse = jnp.log(jnp.sum(jnp.exp(s), axis=1, keepdims=True))
    o_ref[...] = s - lse


def kernel(X_fixed, W_ih, W_hh, b_ih, b_hh, W_fc, b_fc):
    del W_hh
    return pl.pallas_call(
        _fwd,
        grid=(1,),
        in_specs=[
            pl.BlockSpec((1, B, D), lambda i: (0, 0, 0)),
            pl.BlockSpec((D, 3 * H), lambda i: (0, 0)),
            pl.BlockSpec((1, 3 * H), lambda i: (0, 0)),
            pl.BlockSpec((1, 3 * H), lambda i: (0, 0)),
            pl.BlockSpec((H, T), lambda i: (0, 0)),
            pl.BlockSpec((1, T), lambda i: (0, 0)),
        ],
        out_specs=pl.BlockSpec((B, T), lambda i: (0, 0)),
        out_shape=jax.ShapeDtypeStruct((B, T), jnp.float32),
    )(
        X_fixed,
        W_ih.T,
        b_ih.reshape(1, 3 * H),
        b_hh.reshape(1, 3 * H),
        W_fc.T,
        b_fc.reshape(1, T),
    )


# speedup vs baseline: 1126.7948x; 1.7275x over previous
import jax
import jax.numpy as jnp
from jax.experimental import pallas as pl

S, B, D, H, T = 2048, 16, 256, 128, 10


def _fwd(x0_ref, wih_ref, bih_ref, bhh_ref, wfc_ref, bfc_ref, o_ref):
    x0 = x0_ref[0]
    gi = jax.lax.dot_general(x0, wih_ref[...], (((1,), (1,)), ((), ())),
                             preferred_element_type=jnp.float32)
    gi = gi + bih_ref[...]
    bhh = bhh_ref[...]
    r = jax.nn.sigmoid(gi[:, :H] + bhh[:, :H])
    z = jax.nn.sigmoid(gi[:, H:2 * H] + bhh[:, H:2 * H])
    n = jnp.tanh(gi[:, 2 * H:] + r * bhh[:, 2 * H:])
    h1 = (1.0 - z) * n
    logits = jax.lax.dot_general(h1, wfc_ref[...], (((1,), (1,)), ((), ())),
                                 preferred_element_type=jnp.float32)
    logits = logits + bfc_ref[...]
    m = jnp.max(logits, axis=1, keepdims=True)
    s = logits - m
    lse = jnp.log(jnp.sum(jnp.exp(s), axis=1, keepdims=True))
    o_ref[...] = s - lse


def kernel(X_fixed, W_ih, W_hh, b_ih, b_hh, W_fc, b_fc):
    del W_hh
    return pl.pallas_call(
        _fwd,
        grid=(1,),
        in_specs=[
            pl.BlockSpec((1, B, D), lambda i: (0, 0, 0)),
            pl.BlockSpec((3 * H, D), lambda i: (0, 0)),
            pl.BlockSpec((1, 3 * H), lambda i: (0, 0)),
            pl.BlockSpec((1, 3 * H), lambda i: (0, 0)),
            pl.BlockSpec((T, H), lambda i: (0, 0)),
            pl.BlockSpec((1, T), lambda i: (0, 0)),
        ],
        out_specs=pl.BlockSpec((B, T), lambda i: (0, 0)),
        out_shape=jax.ShapeDtypeStruct((B, T), jnp.float32),
    )(
        X_fixed,
        W_ih,
        b_ih.reshape(1, 3 * H),
        b_hh.reshape(1, 3 * H),
        W_fc,
        b_fc.reshape(1, T),
    )


# speedup vs baseline: 2463.8839x; 2.1866x over previous
import jax
import jax.numpy as jnp
from jax.experimental import pallas as pl

S, B, D, H, T = 2048, 16, 256, 128, 10


def _fwd(x0_ref, wih_ref, bih_ref, bhh_ref, wfc_ref, bfc_ref, o_ref):
    x0 = x0_ref[0]
    gi = jax.lax.dot_general(x0, wih_ref[...], (((1,), (1,)), ((), ())),
                             preferred_element_type=jnp.float32)
    gi = gi + bih_ref[...]
    bhh = bhh_ref[...]
    r = jax.nn.sigmoid(gi[:, :H] + bhh[:H])
    z = jax.nn.sigmoid(gi[:, H:2 * H] + bhh[H:2 * H])
    n = jnp.tanh(gi[:, 2 * H:] + r * bhh[2 * H:])
    h1 = (1.0 - z) * n
    logits = jax.lax.dot_general(h1, wfc_ref[...], (((1,), (1,)), ((), ())),
                                 preferred_element_type=jnp.float32)
    logits = logits + bfc_ref[...]
    m = jnp.max(logits, axis=1, keepdims=True)
    s = logits - m
    lse = jnp.log(jnp.sum(jnp.exp(s), axis=1, keepdims=True))
    o_ref[...] = s - lse


def kernel(X_fixed, W_ih, W_hh, b_ih, b_hh, W_fc, b_fc):
    del W_hh
    return pl.pallas_call(
        _fwd,
        grid=(1,),
        in_specs=[
            pl.BlockSpec((1, B, D), lambda i: (0, 0, 0)),
            pl.BlockSpec((3 * H, D), lambda i: (0, 0)),
            pl.BlockSpec((3 * H,), lambda i: (0,)),
            pl.BlockSpec((3 * H,), lambda i: (0,)),
            pl.BlockSpec((T, H), lambda i: (0, 0)),
            pl.BlockSpec((T,), lambda i: (0,)),
        ],
        out_specs=pl.BlockSpec((B, T), lambda i: (0, 0)),
        out_shape=jax.ShapeDtypeStruct((B, T), jnp.float32),
    )(X_fixed, W_ih, b_ih, b_hh, W_fc, b_fc)
